# baseline (speedup 1.0000x reference)
import functools

import jax
import jax.numpy as jnp
from jax import lax
from jax.experimental import pallas as pl
from jax.experimental.pallas import tpu as pltpu
from jax.experimental.pallas import tpu_sc as plsc

BATCH = 4096
SEQ_LEN = 200
EMBED_DIM = 64
CHUNK = 100
NCHUNK = SEQ_LEN // CHUNK
NC = 2
NS = 16
NW = NC * NS
SPW = BATCH // NW
LANES = 16
DCH = EMBED_DIM // LANES


def _fire(emb_hbm, idx_v, s, buf, sem):
  pltpu.async_copy(emb_hbm.at[idx_v.at[s, 0]], buf.at[pl.ds(0, CHUNK)], sem)
  pltpu.async_copy(emb_hbm.at[idx_v.at[s, 1]], buf.at[pl.ds(CHUNK, CHUNK)],
                   sem)


def _drain(emb_hbm, buf, sem):
  pltpu.make_async_copy(emb_hbm.at[pl.ds(0, SEQ_LEN)], buf, sem).wait()


def _accum(buf, acc_v, s):
  zero = jnp.zeros((LANES,), jnp.float32)

  def rbody(r, carry):
    a0, a1, a2, a3 = carry
    for k in range(4):
      row = 4 * r + k
      a0 = a0 + buf[row, pl.ds(0 * LANES, LANES)]
      a1 = a1 + buf[row, pl.ds(1 * LANES, LANES)]
      a2 = a2 + buf[row, pl.ds(2 * LANES, LANES)]
      a3 = a3 + buf[row, pl.ds(3 * LANES, LANES)]
    return (a0, a1, a2, a3)

  a0, a1, a2, a3 = lax.fori_loop(0, SEQ_LEN // 4, rbody,
                                 (zero, zero, zero, zero))
  acc_v[s, pl.ds(0 * LANES, LANES)] = a0
  acc_v[s, pl.ds(1 * LANES, LANES)] = a1
  acc_v[s, pl.ds(2 * LANES, LANES)] = a2
  acc_v[s, pl.ds(3 * LANES, LANES)] = a3


@functools.partial(
    pl.kernel,
    mesh=plsc.VectorSubcoreMesh(core_axis_name="c", subcore_axis_name="s"),
    out_type=jax.ShapeDtypeStruct((BATCH, EMBED_DIM), jnp.float32),
    compiler_params=pltpu.CompilerParams(use_tc_tiling_on_sc=False),
    scratch_types=[
        pltpu.VMEM((SPW, NCHUNK, CHUNK), jnp.int32),
        pltpu.VMEM((SEQ_LEN, EMBED_DIM), jnp.float32),
        pltpu.VMEM((SEQ_LEN, EMBED_DIM), jnp.float32),
        pltpu.VMEM((SPW, EMBED_DIM), jnp.float32),
        pltpu.SemaphoreType.DMA,
        pltpu.SemaphoreType.DMA,
    ],
)
def _pool_sum(x_hbm, emb_hbm, m_hbm, idx_v, buf_a, buf_b, acc_v, sem_a,
              sem_b):
  wid = lax.axis_index("s") * NC + lax.axis_index("c")
  pltpu.sync_copy(x_hbm.at[wid], idx_v)

  _fire(emb_hbm, idx_v, 0, buf_a, sem_a)

  def body(i, _):
    s = 2 * i

    @pl.when(s + 1 < SPW)
    def _():
      _fire(emb_hbm, idx_v, s + 1, buf_b, sem_b)

    _drain(emb_hbm, buf_a, sem_a)
    _accum(buf_a, acc_v, s)

    @pl.when(s + 2 < SPW)
    def _():
      _fire(emb_hbm, idx_v, s + 2, buf_a, sem_a)

    _drain(emb_hbm, buf_b, sem_b)
    _accum(buf_b, acc_v, s + 1)
    return 0

  lax.fori_loop(0, SPW // 2, body, 0)
  pltpu.sync_copy(acc_v, m_hbm.at[pl.ds(wid * SPW, SPW)])


def _mlp_body(m_ref, w1_ref, b1_ref, w2_ref, b2_ref, out_ref):
  m = m_ref[...] * (1.0 / SEQ_LEN)
  h = jnp.dot(m, w1_ref[...], preferred_element_type=jnp.float32)
  h = jnp.maximum(h + b1_ref[...], 0.0)
  out_ref[...] = (
      jnp.dot(h, w2_ref[...], preferred_element_type=jnp.float32)
      + b2_ref[...])


def _mlp(m_sum, W1, b1, W2p, b2p):
  blk = 512
  in_features = W1.shape[1]
  pad_cols = W2p.shape[1]
  return pl.pallas_call(
      _mlp_body,
      grid=(BATCH // blk,),
      in_specs=[
          pl.BlockSpec((blk, EMBED_DIM), lambda i: (i, 0)),
          pl.BlockSpec((EMBED_DIM, in_features), lambda i: (0, 0)),
          pl.BlockSpec((1, in_features), lambda i: (0, 0)),
          pl.BlockSpec((in_features, pad_cols), lambda i: (0, 0)),
          pl.BlockSpec((1, pad_cols), lambda i: (0, 0)),
      ],
      out_specs=pl.BlockSpec((blk, pad_cols), lambda i: (i, 0)),
      out_shape=jax.ShapeDtypeStruct((BATCH, pad_cols), jnp.float32),
  )(m_sum, W1, b1.reshape(1, -1), W2p, b2p.reshape(1, -1))


def kernel(x, emb, W1, b1, W2, b2):
  num_class = W2.shape[1]
  x_r = x.reshape(NW, SPW, NCHUNK, CHUNK)
  m_sum = _pool_sum(x_r, emb)
  pad_cols = 128
  W2p = jnp.pad(W2, ((0, 0), (0, pad_cols - num_class)))
  b2p = jnp.pad(b2, (0, pad_cols - num_class))
  out = _mlp(m_sum, W1, b1, W2p, b2p)
  return out[:, :num_class]
